# baseline (speedup 1.0000x reference)
import functools

import numpy as np
import jax
import jax.numpy as jnp
from jax import lax
from jax.experimental import pallas as pl
from jax.experimental.pallas import tpu as pltpu
from jax.experimental.pallas import tpu_sc as plsc

_B, _C, _H, _W = 8, 64, 32, 32
_HW = _H * _W
_NEG = 256
_TEMP = 2.0
_FACTOR = 0.8
_EPS = 1e-8
_PB = 256
_NPB = _HW // _PB
_NTILES = 32
_RPT = _B * _HW // _NTILES
_CHUNK = 16
_NCHUNKS = _RPT // _CHUNK
_NVEC = _NEG // 16


_U32 = np.uint32


def _tf2x32(k1, k2, x0, x1):
    rot = lambda x, d: ((x << _U32(d)) | (x >> _U32(32 - d))).astype(_U32)
    ks = [k1, k2, (k1 ^ k2 ^ _U32(0x1BD11BDA)).astype(_U32)]
    rr = [[13, 15, 26, 6], [17, 29, 16, 24]]
    x0 = (x0 + ks[0]).astype(_U32)
    x1 = (x1 + ks[1]).astype(_U32)
    for i in range(5):
        for r in rr[i % 2]:
            x0 = (x0 + x1).astype(_U32)
            x1 = rot(x1, r)
            x1 = (x0 ^ x1).astype(_U32)
        x0 = (x0 + ks[(i + 1) % 3]).astype(_U32)
        x1 = (x1 + ks[(i + 2) % 3] + _U32(i + 1)).astype(_U32)
    return x0, x1


def _tf_split2(key):
    b1, b2 = _tf2x32(key[0], key[1], np.zeros(2, _U32), np.arange(2, dtype=_U32))
    return np.stack([b1, b2], axis=1)


def _tf_fold_in(key, i):
    w0, w1 = _tf2x32(key[0], key[1], _U32(0), _U32(i))
    return np.array([w0, w1], _U32)


def _tf_randint_pow2(key, n, span):
    _, k2 = _tf_split2(key)
    counts = np.arange(n, dtype=np.uint64)
    b1, b2 = _tf2x32(k2[0], k2[1],
                     (counts >> np.uint64(32)).astype(_U32),
                     (counts & np.uint64(0xFFFFFFFF)).astype(_U32))
    return ((b1 ^ b2) % _U32(span)).astype(np.int32)


def _build_constants():
    idx = np.zeros((_B, _HW, _NEG), np.int32)
    eucw = np.zeros((_B, _HW), np.float32)
    hh = np.arange(_HW) // _W
    ww = np.arange(_HW) % _W
    cnt = np.zeros((_B, _HW, _HW), np.float32)
    rowbase = np.arange(_HW, dtype=np.int64)[:, None] * _HW
    base_key = np.array([0, 42], _U32)
    for i in range(_B):
        kk = _tf_fold_in(base_key, i)
        ka, kb = _tf_split2(kk)
        nh = _tf_randint_pow2(ka, _HW * _NEG, _H).reshape(_HW, _NEG)
        nw = _tf_randint_pow2(kb, _HW * _NEG, _W).reshape(_HW, _NEG)
        idx[i] = nh * _W + nw
        d2 = (hh[:, None] - nh) ** 2 + (ww[:, None] - nw) ** 2
        eucw[i] = np.sqrt(d2.sum(1)) / np.sqrt((_H - 1.0) ** 2 + (_W - 1.0) ** 2)
        flat = (rowbase + idx[i]).ravel()
        cnt[i] = np.bincount(flat, minlength=_HW * _HW).reshape(_HW, _HW)
    return idx, eucw, cnt


_IDX_NP, _EUCW_NP, _CNT_NP = _build_constants()
_raw = _IDX_NP.reshape(_B * _HW, _NEG).astype(np.int64)
_word = (np.arange(_B * _HW, dtype=np.int64) % _CHUNK)[:, None] * (_HW // 2) \
    + (_raw >> 1)
_packed = (_word | ((1 - (_raw & 1)) << 31)).astype(np.uint64)
_IDX = np.ascontiguousarray(
    _packed.reshape(-1).astype(np.uint32).view(np.int32))
_EUCW = np.ascontiguousarray((_EUCW_NP * _FACTOR).reshape(_B * _NPB, 1, _PB))
_CNT = _CNT_NP


def _prep_body(z1_ref, z2_ref, imgf_ref, imgp_ref, cnt_ref, euc_ref,
               v_ref, p0_ref):
    z1 = z1_ref[0]
    z2 = z2_ref[0]
    imgf = imgf_ref[...]
    imgp = imgp_ref[...]
    cntb = cnt_ref[0]
    euc = euc_ref[0]

    n1sq = jnp.sum(z1 * z1, axis=0, keepdims=True)
    n2sq = jnp.sum(z2 * z2, axis=0, keepdims=True)

    g = lax.dot_general(z1, z2, (((0,), (0,)), ((), ())),
                        preferred_element_type=jnp.float32)

    s_row = jnp.sum(imgf * imgf, axis=0, keepdims=True)
    m4 = jnp.concatenate([s_row, imgf], axis=0)
    t4 = lax.dot_general(m4, cntb, (((1,), (1,)), ((), ())),
                         preferred_element_type=jnp.float32)
    s_p = jnp.sum(imgp * imgp, axis=0, keepdims=True)
    rgbsq = _NEG * s_p + t4[0:1, :] - 2.0 * jnp.sum(
        imgp * t4[1:4, :], axis=0, keepdims=True)
    rgb = jnp.sqrt(jnp.maximum(rgbsq, 0.0))
    w_row = euc + rgb * ((1.0 - _FACTOR) / np.sqrt(3.0))

    n1col = jnp.transpose(jnp.sqrt(n1sq))
    wcol = jnp.transpose(w_row)
    den = jnp.maximum(n1col * jnp.sqrt(n2sq), _EPS)
    v_ref[0] = jnp.minimum(jnp.abs(g) * wcol / den, 1.0).astype(jnp.bfloat16)

    ratio = jnp.minimum(n1sq / jnp.maximum(n1sq, _EPS), 1.0)
    p0_ref[...] = jnp.full((1, 1, 128), jnp.sum(ratio), jnp.float32)


_prep = pl.pallas_call(
    _prep_body,
    grid=(_B, _NPB),
    in_specs=[
        pl.BlockSpec((1, _C, _PB), lambda i, pb: (i, 0, pb)),
        pl.BlockSpec((1, _C, _HW), lambda i, pb: (i, 0, 0)),
        pl.BlockSpec((3, _HW), lambda i, pb: (0, 0)),
        pl.BlockSpec((3, _PB), lambda i, pb: (0, pb)),
        pl.BlockSpec((1, _PB, _HW), lambda i, pb: (i, pb, 0)),
        pl.BlockSpec((1, 1, _PB), lambda i, pb: (i * _NPB + pb, 0, 0)),
    ],
    out_specs=[
        pl.BlockSpec((1, _PB, _HW), lambda i, pb: (i, pb, 0)),
        pl.BlockSpec((1, 1, 128), lambda i, pb: (i * _NPB + pb, 0, 0)),
    ],
    out_shape=[
        jax.ShapeDtypeStruct((_B, _HW, _HW), jnp.bfloat16),
        jax.ShapeDtypeStruct((_B * _NPB, 1, 128), jnp.float32),
    ],
)


_mesh = plsc.VectorSubcoreMesh(core_axis_name="c", subcore_axis_name="s")


_VB = _CHUNK * _HW // 2
_IB = _CHUNK * _NEG


@functools.partial(
    pl.kernel,
    mesh=_mesh,
    out_type=jax.ShapeDtypeStruct((_NTILES * _NEG,), jnp.float32),
    scratch_types=[
        pltpu.VMEM((2 * _VB,), jnp.int32),
        pltpu.VMEM((2 * _IB,), jnp.int32),
        pltpu.VMEM((_NEG,), jnp.float32),
        pltpu.SemaphoreType.DMA,
        pltpu.SemaphoreType.DMA,
        pltpu.SemaphoreType.DMA,
        pltpu.SemaphoreType.DMA,
    ],
    compiler_params=pltpu.CompilerParams(needs_layout_passes=False),
)
def _sc_gather(v_hbm, idx_hbm, out_hbm, vbuf, ibuf, accbuf,
               sv0, si0, sv1, si1):
    wid = lax.axis_index("s") * 2 + lax.axis_index("c")
    base = wid * _RPT
    sems = ((sv0, si0), (sv1, si1))

    def copies(ch, slot):
        row0 = base + ch * _CHUNK
        sv, si = sems[slot]
        return (
            pltpu.make_async_copy(
                v_hbm.at[pl.ds(row0 * (_HW // 2), _VB)],
                vbuf.at[pl.ds(slot * _VB, _VB)], sv),
            pltpu.make_async_copy(
                idx_hbm.at[pl.ds(row0 * _NEG, _IB)],
                ibuf.at[pl.ds(slot * _IB, _IB)], si),
        )

    def issue(ch, slot):
        for c in copies(ch, slot):
            c.start()

    def wait(ch, slot):
        for c in copies(ch, slot):
            c.wait()

    def compute(slot, accs):
        accs = list(accs)
        for r in range(_CHUNK):
            for k in range(_NVEC):
                pw = ibuf[pl.ds(slot * _IB + r * _NEG + k * 16, 16)]
                widx = pw & jnp.int32(0x1FFF)
                sh = lax.shift_right_logical(pw, jnp.int32(27))
                w = plsc.load_gather(vbuf.at[pl.ds(slot * _VB, _VB)], [widx])
                fb = lax.shift_left(w, sh) & jnp.int32(-65536)
                accs[k] = accs[k] + plsc.bitcast(fb, jnp.float32)
        return tuple(accs)

    issue(0, 0)

    def pair_body(c2, accs):
        c0 = 2 * c2
        issue(c0 + 1, 1)
        wait(c0, 0)
        accs = compute(0, accs)

        @pl.when(c2 < _NCHUNKS // 2 - 1)
        def _():
            issue(c0 + 2, 0)

        wait(c0 + 1, 1)
        return compute(1, accs)

    accs = lax.fori_loop(
        0, _NCHUNKS // 2, pair_body,
        tuple(jnp.zeros((16,), jnp.float32) for _ in range(_NVEC)))
    for k in range(_NVEC):
        accbuf[pl.ds(k * 16, 16)] = accs[k]
    pltpu.sync_copy(accbuf, out_hbm.at[pl.ds(wid * _NEG, _NEG)])


def _final_body(sp_ref, p0_ref, out_ref):
    sp = sp_ref[...]
    p0p = p0_ref[...]
    s = jnp.sum(sp, axis=1)
    p0 = jnp.sum(p0p, axis=1)[:, 0:1] / _HW
    pred = s / _HW / _TEMP
    negterm = jnp.maximum(jnp.log(1.0 - pred), -100.0)
    safe = jnp.where(p0 > 0.0, p0, 0.5)
    posterm = jnp.where(p0 > 0.0, jnp.maximum(jnp.log(safe), -100.0), -100.0)
    lossi = -(posterm + jnp.sum(negterm, axis=1, keepdims=True)) / (_NEG + 1.0)
    loss = jnp.sum(lossi) / _B
    o2 = jnp.sum(p0) / _B
    o3 = jnp.sum(pred) / _NEG * _TEMP / _B
    lane = lax.broadcasted_iota(jnp.int32, (1, 128), 1)
    out_ref[...] = jnp.where(
        lane == 0, loss, jnp.where(lane == 1, o2, jnp.where(lane == 2, o3, 0.0)))


_final = pl.pallas_call(
    _final_body,
    out_shape=jax.ShapeDtypeStruct((1, 128), jnp.float32),
)


def kernel(views_1, views_2, img):
    z1 = views_1.reshape(_B, _C, _HW)
    z2 = views_2.reshape(_B, _C, _HW)
    imgf = img.reshape(3, _HW)
    v, p0 = _prep(z1, z2, imgf, imgf, _CNT, _EUCW)
    v32 = lax.bitcast_convert_type(
        v.reshape(_B * _HW * _HW // 2, 2), jnp.int32)
    sp = _sc_gather(v32, _IDX)
    out = _final(sp.reshape(_B, _NTILES // _B, _NEG),
                 p0.reshape(_B, _NPB, 128))
    return (out[0, 0], out[0, 1], out[0, 2])


# speedup vs baseline: 26.8642x; 26.8642x over previous
import functools

import numpy as np
import jax
import jax.numpy as jnp
from jax import lax
from jax.experimental import pallas as pl
from jax.experimental.pallas import tpu as pltpu
from jax.experimental.pallas import tpu_sc as plsc

_B, _C, _H, _W = 8, 64, 32, 32
_HW = _H * _W
_NEG = 256
_TEMP = 2.0
_FACTOR = 0.8
_EPS = 1e-8
_PB = 256
_NPB = _HW // _PB
_NTILES = 32
_RPT = _B * _HW // _NTILES
_CHUNK = 16
_NCHUNKS = _RPT // _CHUNK
_NVEC = _NEG // 16


_U32 = np.uint32


def _tf2x32(k1, k2, x0, x1):
    rot = lambda x, d: ((x << _U32(d)) | (x >> _U32(32 - d))).astype(_U32)
    ks = [k1, k2, (k1 ^ k2 ^ _U32(0x1BD11BDA)).astype(_U32)]
    rr = [[13, 15, 26, 6], [17, 29, 16, 24]]
    x0 = (x0 + ks[0]).astype(_U32)
    x1 = (x1 + ks[1]).astype(_U32)
    for i in range(5):
        for r in rr[i % 2]:
            x0 = (x0 + x1).astype(_U32)
            x1 = rot(x1, r)
            x1 = (x0 ^ x1).astype(_U32)
        x0 = (x0 + ks[(i + 1) % 3]).astype(_U32)
        x1 = (x1 + ks[(i + 2) % 3] + _U32(i + 1)).astype(_U32)
    return x0, x1


def _tf_split2(key):
    b1, b2 = _tf2x32(key[0], key[1], np.zeros(2, _U32), np.arange(2, dtype=_U32))
    return np.stack([b1, b2], axis=1)


def _tf_fold_in(key, i):
    w0, w1 = _tf2x32(key[0], key[1], _U32(0), _U32(i))
    return np.array([w0, w1], _U32)


def _tf_randint_pow2(key, n, span):
    _, k2 = _tf_split2(key)
    counts = np.arange(n, dtype=np.uint64)
    b1, b2 = _tf2x32(k2[0], k2[1],
                     (counts >> np.uint64(32)).astype(_U32),
                     (counts & np.uint64(0xFFFFFFFF)).astype(_U32))
    return ((b1 ^ b2) % _U32(span)).astype(np.int32)


def _build_constants():
    idx = np.zeros((_B, _HW, _NEG), np.int32)
    eucw = np.zeros((_B, _HW), np.float32)
    hh = np.arange(_HW) // _W
    ww = np.arange(_HW) % _W
    cnt = np.zeros((_B, _HW, _HW), np.float32)
    rowbase = np.arange(_HW, dtype=np.int64)[:, None] * _HW
    base_key = np.array([0, 42], _U32)
    for i in range(_B):
        kk = _tf_fold_in(base_key, i)
        ka, kb = _tf_split2(kk)
        nh = _tf_randint_pow2(ka, _HW * _NEG, _H).reshape(_HW, _NEG)
        nw = _tf_randint_pow2(kb, _HW * _NEG, _W).reshape(_HW, _NEG)
        idx[i] = nh * _W + nw
        d2 = (hh[:, None] - nh) ** 2 + (ww[:, None] - nw) ** 2
        eucw[i] = np.sqrt(d2.sum(1)) / np.sqrt((_H - 1.0) ** 2 + (_W - 1.0) ** 2)
        flat = (rowbase + idx[i]).ravel()
        cnt[i] = np.bincount(flat, minlength=_HW * _HW).reshape(_HW, _HW)
    return idx, eucw, cnt


_IDX_NP, _EUCW_NP, _CNT_NP = _build_constants()
_raw = _IDX_NP.reshape(_B * _HW, _NEG).astype(np.int64)
_word = (np.arange(_B * _HW, dtype=np.int64) % _CHUNK)[:, None] * (_HW // 2) \
    + (_raw & (_HW // 2 - 1))
_packed = (_word | ((_raw < _HW // 2).astype(np.int64) << 31)).astype(np.uint64)
_IDX = np.ascontiguousarray(
    _packed.reshape(-1).astype(np.uint32).view(np.int32))
_EUCW = np.ascontiguousarray((_EUCW_NP * _FACTOR).reshape(_B * _NPB, 1, _PB))
_CNT = _CNT_NP


def _prep_body(z1_ref, z2_ref, imgf_ref, imgp_ref, cnt_ref, euc_ref,
               v_ref, p0_ref):
    z1 = z1_ref[0]
    z2 = z2_ref[0]
    imgf = imgf_ref[...]
    imgp = imgp_ref[...]
    cntb = cnt_ref[0]
    euc = euc_ref[0]

    n1sq = jnp.sum(z1 * z1, axis=0, keepdims=True)
    n2sq = jnp.sum(z2 * z2, axis=0, keepdims=True)

    g = lax.dot_general(z1, z2, (((0,), (0,)), ((), ())),
                        preferred_element_type=jnp.float32)

    s_row = jnp.sum(imgf * imgf, axis=0, keepdims=True)
    m4 = jnp.concatenate([s_row, imgf], axis=0)
    t4 = lax.dot_general(m4, cntb, (((1,), (1,)), ((), ())),
                         preferred_element_type=jnp.float32)
    s_p = jnp.sum(imgp * imgp, axis=0, keepdims=True)
    rgbsq = _NEG * s_p + t4[0:1, :] - 2.0 * jnp.sum(
        imgp * t4[1:4, :], axis=0, keepdims=True)
    rgb = jnp.sqrt(jnp.maximum(rgbsq, 0.0))
    w_row = euc + rgb * ((1.0 - _FACTOR) / np.sqrt(3.0))

    n1col = jnp.transpose(jnp.sqrt(n1sq))
    wcol = jnp.transpose(w_row)
    den = jnp.maximum(n1col * jnp.sqrt(n2sq), _EPS)
    v = jnp.minimum(jnp.abs(g) * wcol / den, 1.0)

    def _bf16_bits(x):
        b = lax.bitcast_convert_type(x, jnp.int32)
        return lax.shift_right_logical(
            b + 0x7FFF + (lax.shift_right_logical(b, 16) & 1), 16)

    lo = _bf16_bits(v[:, : _HW // 2])
    hi = _bf16_bits(v[:, _HW // 2:])
    v_ref[0] = lo | lax.shift_left(hi, 16)

    ratio = jnp.minimum(n1sq / jnp.maximum(n1sq, _EPS), 1.0)
    p0_ref[...] = jnp.full((1, 1, 128), jnp.sum(ratio), jnp.float32)


_prep = pl.pallas_call(
    _prep_body,
    grid=(_B, _NPB),
    in_specs=[
        pl.BlockSpec((1, _C, _PB), lambda i, pb: (i, 0, pb)),
        pl.BlockSpec((1, _C, _HW), lambda i, pb: (i, 0, 0)),
        pl.BlockSpec((3, _HW), lambda i, pb: (0, 0)),
        pl.BlockSpec((3, _PB), lambda i, pb: (0, pb)),
        pl.BlockSpec((1, _PB, _HW), lambda i, pb: (i, pb, 0)),
        pl.BlockSpec((1, 1, _PB), lambda i, pb: (i * _NPB + pb, 0, 0)),
    ],
    out_specs=[
        pl.BlockSpec((1, _PB, _HW // 2), lambda i, pb: (i, pb, 0)),
        pl.BlockSpec((1, 1, 128), lambda i, pb: (i * _NPB + pb, 0, 0)),
    ],
    out_shape=[
        jax.ShapeDtypeStruct((_B, _HW, _HW // 2), jnp.int32),
        jax.ShapeDtypeStruct((_B * _NPB, 1, 128), jnp.float32),
    ],
)


_mesh = plsc.VectorSubcoreMesh(core_axis_name="c", subcore_axis_name="s")


_VB = _CHUNK * _HW // 2
_IB = _CHUNK * _NEG


@functools.partial(
    pl.kernel,
    mesh=_mesh,
    out_type=jax.ShapeDtypeStruct((_NTILES * _NEG,), jnp.float32),
    scratch_types=[
        pltpu.VMEM((2 * _VB,), jnp.int32),
        pltpu.VMEM((2 * _IB,), jnp.int32),
        pltpu.VMEM((_NEG,), jnp.float32),
        pltpu.SemaphoreType.DMA,
        pltpu.SemaphoreType.DMA,
        pltpu.SemaphoreType.DMA,
        pltpu.SemaphoreType.DMA,
    ],
    compiler_params=pltpu.CompilerParams(needs_layout_passes=False),
)
def _sc_gather(v_hbm, idx_hbm, out_hbm, vbuf, ibuf, accbuf,
               sv0, si0, sv1, si1):
    wid = lax.axis_index("s") * 2 + lax.axis_index("c")
    base = wid * _RPT
    sems = ((sv0, si0), (sv1, si1))

    def copies(ch, slot):
        row0 = base + ch * _CHUNK
        sv, si = sems[slot]
        return (
            pltpu.make_async_copy(
                v_hbm.at[pl.ds(row0 * (_HW // 2), _VB)],
                vbuf.at[pl.ds(slot * _VB, _VB)], sv),
            pltpu.make_async_copy(
                idx_hbm.at[pl.ds(row0 * _NEG, _IB)],
                ibuf.at[pl.ds(slot * _IB, _IB)], si),
        )

    def issue(ch, slot):
        for c in copies(ch, slot):
            c.start()

    def wait(ch, slot):
        for c in copies(ch, slot):
            c.wait()

    def compute(slot, accs):
        accs = list(accs)
        for r in range(_CHUNK):
            for k in range(_NVEC):
                pw = ibuf[pl.ds(slot * _IB + r * _NEG + k * 16, 16)]
                widx = pw & jnp.int32(0x1FFF)
                sh = lax.shift_right_logical(pw, jnp.int32(27))
                w = plsc.load_gather(vbuf.at[pl.ds(slot * _VB, _VB)], [widx])
                fb = lax.shift_left(w, sh) & jnp.int32(-65536)
                accs[k] = accs[k] + plsc.bitcast(fb, jnp.float32)
        return tuple(accs)

    issue(0, 0)

    def pair_body(c2, accs):
        c0 = 2 * c2
        issue(c0 + 1, 1)
        wait(c0, 0)
        accs = compute(0, accs)

        @pl.when(c2 < _NCHUNKS // 2 - 1)
        def _():
            issue(c0 + 2, 0)

        wait(c0 + 1, 1)
        return compute(1, accs)

    accs = lax.fori_loop(
        0, _NCHUNKS // 2, pair_body,
        tuple(jnp.zeros((16,), jnp.float32) for _ in range(_NVEC)))
    for k in range(_NVEC):
        accbuf[pl.ds(k * 16, 16)] = accs[k]
    pltpu.sync_copy(accbuf, out_hbm.at[pl.ds(wid * _NEG, _NEG)])


def _final_body(sp_ref, p0_ref, out_ref):
    sp = sp_ref[...]
    p0p = p0_ref[...]
    s = jnp.sum(sp, axis=1)
    p0 = jnp.sum(p0p, axis=1)[:, 0:1] / _HW
    pred = s / _HW / _TEMP
    negterm = jnp.maximum(jnp.log(1.0 - pred), -100.0)
    safe = jnp.where(p0 > 0.0, p0, 0.5)
    posterm = jnp.where(p0 > 0.0, jnp.maximum(jnp.log(safe), -100.0), -100.0)
    lossi = -(posterm + jnp.sum(negterm, axis=1, keepdims=True)) / (_NEG + 1.0)
    loss = jnp.sum(lossi) / _B
    o2 = jnp.sum(p0) / _B
    o3 = jnp.sum(pred) / _NEG * _TEMP / _B
    lane = lax.broadcasted_iota(jnp.int32, (1, 128), 1)
    out_ref[...] = jnp.where(
        lane == 0, loss, jnp.where(lane == 1, o2, jnp.where(lane == 2, o3, 0.0)))


_final = pl.pallas_call(
    _final_body,
    out_shape=jax.ShapeDtypeStruct((1, 128), jnp.float32),
)


def kernel(views_1, views_2, img):
    z1 = views_1.reshape(_B, _C, _HW)
    z2 = views_2.reshape(_B, _C, _HW)
    imgf = img.reshape(3, _HW)
    v, p0 = _prep(z1, z2, imgf, imgf, _CNT, _EUCW)
    sp = _sc_gather(v.reshape(_B * _HW * _HW // 2), _IDX)
    out = _final(sp.reshape(_B, _NTILES // _B, _NEG),
                 p0.reshape(_B, _NPB, 128))
    return (out[0, 0], out[0, 1], out[0, 2])


# speedup vs baseline: 27.7964x; 1.0347x over previous
import functools

import numpy as np
import jax
import jax.numpy as jnp
from jax import lax
from jax.experimental import pallas as pl
from jax.experimental.pallas import tpu as pltpu
from jax.experimental.pallas import tpu_sc as plsc

_B, _C, _H, _W = 8, 64, 32, 32
_HW = _H * _W
_NEG = 256
_TEMP = 2.0
_FACTOR = 0.8
_EPS = 1e-8
_PB = 256
_NPB = _HW // _PB
_NTILES = 32
_RPT = _B * _HW // _NTILES
_CHUNK = 16
_NCHUNKS = _RPT // _CHUNK
_NVEC = _NEG // 16


_U32 = np.uint32


def _tf2x32(k1, k2, x0, x1):
    rot = lambda x, d: ((x << _U32(d)) | (x >> _U32(32 - d))).astype(_U32)
    ks = [k1, k2, (k1 ^ k2 ^ _U32(0x1BD11BDA)).astype(_U32)]
    rr = [[13, 15, 26, 6], [17, 29, 16, 24]]
    x0 = (x0 + ks[0]).astype(_U32)
    x1 = (x1 + ks[1]).astype(_U32)
    for i in range(5):
        for r in rr[i % 2]:
            x0 = (x0 + x1).astype(_U32)
            x1 = rot(x1, r)
            x1 = (x0 ^ x1).astype(_U32)
        x0 = (x0 + ks[(i + 1) % 3]).astype(_U32)
        x1 = (x1 + ks[(i + 2) % 3] + _U32(i + 1)).astype(_U32)
    return x0, x1


def _tf_split2(key):
    b1, b2 = _tf2x32(key[0], key[1], np.zeros(2, _U32), np.arange(2, dtype=_U32))
    return np.stack([b1, b2], axis=1)


def _tf_fold_in(key, i):
    w0, w1 = _tf2x32(key[0], key[1], _U32(0), _U32(i))
    return np.array([w0, w1], _U32)


def _tf_randint_pow2(key, n, span):
    _, k2 = _tf_split2(key)
    counts = np.arange(n, dtype=np.uint64)
    b1, b2 = _tf2x32(k2[0], k2[1],
                     (counts >> np.uint64(32)).astype(_U32),
                     (counts & np.uint64(0xFFFFFFFF)).astype(_U32))
    return ((b1 ^ b2) % _U32(span)).astype(np.int32)


def _build_constants():
    idx = np.zeros((_B, _HW, _NEG), np.int32)
    eucw = np.zeros((_B, _HW), np.float32)
    hh = np.arange(_HW) // _W
    ww = np.arange(_HW) % _W
    cnt = np.zeros((_B, _HW, _HW), np.float32)
    rowbase = np.arange(_HW, dtype=np.int64)[:, None] * _HW
    base_key = np.array([0, 42], _U32)
    for i in range(_B):
        kk = _tf_fold_in(base_key, i)
        ka, kb = _tf_split2(kk)
        nh = _tf_randint_pow2(ka, _HW * _NEG, _H).reshape(_HW, _NEG)
        nw = _tf_randint_pow2(kb, _HW * _NEG, _W).reshape(_HW, _NEG)
        idx[i] = nh * _W + nw
        d2 = (hh[:, None] - nh) ** 2 + (ww[:, None] - nw) ** 2
        eucw[i] = np.sqrt(d2.sum(1)) / np.sqrt((_H - 1.0) ** 2 + (_W - 1.0) ** 2)
        flat = (rowbase + idx[i]).ravel()
        cnt[i] = np.bincount(flat, minlength=_HW * _HW).reshape(_HW, _HW)
    return idx, eucw, cnt


_IDX_NP, _EUCW_NP, _CNT_NP = _build_constants()
_raw = _IDX_NP.reshape(_B * _HW, _NEG).astype(np.int64)
_rr = (np.arange(_B * _HW, dtype=np.int64) % _CHUNK)[:, None]
_x = _raw & (_HW // 2 - 1)
_packed = ((_rr << 13) | _x
           | ((_raw < _HW // 2).astype(np.int64) << 31)).astype(np.uint64)
_IDX = np.ascontiguousarray(
    _packed.reshape(-1).astype(np.uint32).view(np.int32))
_EUCW = np.ascontiguousarray((_EUCW_NP * _FACTOR).reshape(_B * _NPB, 1, _PB))
_CNT = _CNT_NP


def _prep_body(z1_ref, z2_ref, imgf_ref, imgp_ref, cnt_ref, euc_ref,
               v_ref, p0_ref):
    z1 = z1_ref[0]
    z2 = z2_ref[0]
    imgf = imgf_ref[...]
    imgp = imgp_ref[...]
    cntb = cnt_ref[0]
    euc = euc_ref[0]

    n1sq = jnp.sum(z1 * z1, axis=0, keepdims=True)
    n2sq = jnp.sum(z2 * z2, axis=0, keepdims=True)

    g = lax.dot_general(z1, z2, (((0,), (0,)), ((), ())),
                        preferred_element_type=jnp.float32)

    s_row = jnp.sum(imgf * imgf, axis=0, keepdims=True)
    m4 = jnp.concatenate([s_row, imgf], axis=0)
    t4 = lax.dot_general(m4, cntb, (((1,), (1,)), ((), ())),
                         preferred_element_type=jnp.float32)
    s_p = jnp.sum(imgp * imgp, axis=0, keepdims=True)
    rgbsq = _NEG * s_p + t4[0:1, :] - 2.0 * jnp.sum(
        imgp * t4[1:4, :], axis=0, keepdims=True)
    rgb = jnp.sqrt(jnp.maximum(rgbsq, 0.0))
    w_row = euc + rgb * ((1.0 - _FACTOR) / np.sqrt(3.0))

    n1col = jnp.transpose(jnp.sqrt(n1sq))
    wcol = jnp.transpose(w_row)
    den = jnp.maximum(n1col * jnp.sqrt(n2sq), _EPS)
    v = jnp.minimum(jnp.abs(g) * wcol / den, 1.0)

    def _bf16_bits(x):
        b = lax.bitcast_convert_type(x, jnp.int32)
        return lax.shift_right_logical(
            b + 0x7FFF + (lax.shift_right_logical(b, 16) & 1), 16)

    lo = _bf16_bits(v[:, : _HW // 2])
    hi = _bf16_bits(v[:, _HW // 2:])
    v_ref[0] = lo | lax.shift_left(hi, 16)

    ratio = jnp.minimum(n1sq / jnp.maximum(n1sq, _EPS), 1.0)
    p0_ref[...] = jnp.full((1, 1, 128), jnp.sum(ratio), jnp.float32)


_prep = pl.pallas_call(
    _prep_body,
    grid=(_B, _NPB),
    in_specs=[
        pl.BlockSpec((1, _C, _PB), lambda i, pb: (i, 0, pb)),
        pl.BlockSpec((1, _C, _HW), lambda i, pb: (i, 0, 0)),
        pl.BlockSpec((3, _HW), lambda i, pb: (0, 0)),
        pl.BlockSpec((3, _PB), lambda i, pb: (0, pb)),
        pl.BlockSpec((1, _PB, _HW), lambda i, pb: (i, pb, 0)),
        pl.BlockSpec((1, 1, _PB), lambda i, pb: (i * _NPB + pb, 0, 0)),
    ],
    out_specs=[
        pl.BlockSpec((1, _PB, _HW // 2), lambda i, pb: (i, pb, 0)),
        pl.BlockSpec((1, 1, 128), lambda i, pb: (i * _NPB + pb, 0, 0)),
    ],
    out_shape=[
        jax.ShapeDtypeStruct((_B, _HW, _HW // 2), jnp.int32),
        jax.ShapeDtypeStruct((_B * _NPB, 1, 128), jnp.float32),
    ],
)


_mesh = plsc.VectorSubcoreMesh(core_axis_name="c", subcore_axis_name="s")


_VB = _CHUNK * _HW // 2
_IB = _CHUNK * _NEG


@functools.partial(
    pl.kernel,
    mesh=_mesh,
    out_type=jax.ShapeDtypeStruct((_NTILES * _NEG,), jnp.float32),
    scratch_types=[
        pltpu.VMEM((2, _CHUNK, _HW // 2), jnp.int32),
        pltpu.VMEM((2 * _IB,), jnp.int32),
        pltpu.VMEM((_NEG,), jnp.float32),
        pltpu.SemaphoreType.DMA,
        pltpu.SemaphoreType.DMA,
        pltpu.SemaphoreType.DMA,
        pltpu.SemaphoreType.DMA,
    ],
    compiler_params=pltpu.CompilerParams(needs_layout_passes=False),
)
def _sc_gather(v_hbm, idx_hbm, out_hbm, vbuf, ibuf, accbuf,
               sv0, si0, sv1, si1):
    wid = lax.axis_index("s") * 2 + lax.axis_index("c")
    base = wid * _RPT
    sems = ((sv0, si0), (sv1, si1))

    def copies(ch, slot):
        row0 = base + ch * _CHUNK
        sv, si = sems[slot]
        return (
            pltpu.make_async_copy(
                v_hbm.at[pl.ds(row0, _CHUNK)],
                vbuf.at[slot], sv),
            pltpu.make_async_copy(
                idx_hbm.at[pl.ds(row0 * _NEG, _IB)],
                ibuf.at[pl.ds(slot * _IB, _IB)], si),
        )

    def issue(ch, slot):
        for c in copies(ch, slot):
            c.start()

    def wait(ch, slot):
        for c in copies(ch, slot):
            c.wait()

    def compute(slot, accs):
        accs = list(accs)
        for r in range(_CHUNK):
            for k in range(_NVEC):
                pw = ibuf[pl.ds(slot * _IB + r * _NEG + k * 16, 16)]
                rv = lax.shift_right_logical(pw, jnp.int32(13)) & jnp.int32(15)
                cv = pw & jnp.int32(511)
                sh = lax.shift_right_logical(pw, jnp.int32(27)) & jnp.int32(16)
                w = plsc.load_gather(vbuf.at[slot], [rv, cv])
                fb = lax.shift_left(w, sh) & jnp.int32(-65536)
                accs[k] = accs[k] + plsc.bitcast(fb, jnp.float32)
        return tuple(accs)

    issue(0, 0)

    def pair_body(c2, accs):
        c0 = 2 * c2
        issue(c0 + 1, 1)
        wait(c0, 0)
        accs = compute(0, accs)

        @pl.when(c2 < _NCHUNKS // 2 - 1)
        def _():
            issue(c0 + 2, 0)

        wait(c0 + 1, 1)
        return compute(1, accs)

    accs = lax.fori_loop(
        0, _NCHUNKS // 2, pair_body,
        tuple(jnp.zeros((16,), jnp.float32) for _ in range(_NVEC)))
    for k in range(_NVEC):
        accbuf[pl.ds(k * 16, 16)] = accs[k]
    pltpu.sync_copy(accbuf, out_hbm.at[pl.ds(wid * _NEG, _NEG)])


def _final_body(sp_ref, p0_ref, out_ref):
    sp = sp_ref[...]
    p0p = p0_ref[...]
    s = jnp.sum(sp, axis=1)
    p0 = jnp.sum(p0p, axis=1)[:, 0:1] / _HW
    pred = s / _HW / _TEMP
    negterm = jnp.maximum(jnp.log(1.0 - pred), -100.0)
    safe = jnp.where(p0 > 0.0, p0, 0.5)
    posterm = jnp.where(p0 > 0.0, jnp.maximum(jnp.log(safe), -100.0), -100.0)
    lossi = -(posterm + jnp.sum(negterm, axis=1, keepdims=True)) / (_NEG + 1.0)
    loss = jnp.sum(lossi) / _B
    o2 = jnp.sum(p0) / _B
    o3 = jnp.sum(pred) / _NEG * _TEMP / _B
    lane = lax.broadcasted_iota(jnp.int32, (1, 128), 1)
    out_ref[...] = jnp.where(
        lane == 0, loss, jnp.where(lane == 1, o2, jnp.where(lane == 2, o3, 0.0)))


_final = pl.pallas_call(
    _final_body,
    out_shape=jax.ShapeDtypeStruct((1, 128), jnp.float32),
)


def kernel(views_1, views_2, img):
    z1 = views_1.reshape(_B, _C, _HW)
    z2 = views_2.reshape(_B, _C, _HW)
    imgf = img.reshape(3, _HW)
    v, p0 = _prep(z1, z2, imgf, imgf, _CNT, _EUCW)
    sp = _sc_gather(v.reshape(_B * _HW, _HW // 2), _IDX)
    out = _final(sp.reshape(_B, _NTILES // _B, _NEG),
                 p0.reshape(_B, _NPB, 128))
    return (out[0, 0], out[0, 1], out[0, 2])


# speedup vs baseline: 34.8692x; 1.2545x over previous
import functools

import numpy as np
import jax
import jax.numpy as jnp
from jax import lax
from jax.experimental import pallas as pl
from jax.experimental.pallas import tpu as pltpu
from jax.experimental.pallas import tpu_sc as plsc

_B, _C, _H, _W = 8, 64, 32, 32
_HW = _H * _W
_NEG = 256
_TEMP = 2.0
_FACTOR = 0.8
_EPS = 1e-8
_PB = 256
_NPB = _HW // _PB
_NTILES = 32
_RPT = _B * _HW // _NTILES
_CHUNK = 64
_NCHUNKS = _RPT // _CHUNK
_NVEC = _NEG // 16


_U32 = np.uint32


def _tf2x32(k1, k2, x0, x1):
    rot = lambda x, d: ((x << _U32(d)) | (x >> _U32(32 - d))).astype(_U32)
    ks = [k1, k2, (k1 ^ k2 ^ _U32(0x1BD11BDA)).astype(_U32)]
    rr = [[13, 15, 26, 6], [17, 29, 16, 24]]
    x0 = (x0 + ks[0]).astype(_U32)
    x1 = (x1 + ks[1]).astype(_U32)
    for i in range(5):
        for r in rr[i % 2]:
            x0 = (x0 + x1).astype(_U32)
            x1 = rot(x1, r)
            x1 = (x0 ^ x1).astype(_U32)
        x0 = (x0 + ks[(i + 1) % 3]).astype(_U32)
        x1 = (x1 + ks[(i + 2) % 3] + _U32(i + 1)).astype(_U32)
    return x0, x1


def _tf_split2(key):
    b1, b2 = _tf2x32(key[0], key[1], np.zeros(2, _U32), np.arange(2, dtype=_U32))
    return np.stack([b1, b2], axis=1)


def _tf_fold_in(key, i):
    w0, w1 = _tf2x32(key[0], key[1], _U32(0), _U32(i))
    return np.array([w0, w1], _U32)


def _tf_randint_pow2(key, n, span):
    _, k2 = _tf_split2(key)
    counts = np.arange(n, dtype=np.uint64)
    b1, b2 = _tf2x32(k2[0], k2[1],
                     (counts >> np.uint64(32)).astype(_U32),
                     (counts & np.uint64(0xFFFFFFFF)).astype(_U32))
    return ((b1 ^ b2) % _U32(span)).astype(np.int32)


def _build_constants():
    idx = np.zeros((_B, _HW, _NEG), np.int32)
    eucw = np.zeros((_B, _HW), np.float32)
    hh = np.arange(_HW) // _W
    ww = np.arange(_HW) % _W
    cnt = np.zeros((_B, _HW, _HW), np.float32)
    rowbase = np.arange(_HW, dtype=np.int64)[:, None] * _HW
    base_key = np.array([0, 42], _U32)
    for i in range(_B):
        kk = _tf_fold_in(base_key, i)
        ka, kb = _tf_split2(kk)
        nh = _tf_randint_pow2(ka, _HW * _NEG, _H).reshape(_HW, _NEG)
        nw = _tf_randint_pow2(kb, _HW * _NEG, _W).reshape(_HW, _NEG)
        idx[i] = nh * _W + nw
        d2 = (hh[:, None] - nh) ** 2 + (ww[:, None] - nw) ** 2
        eucw[i] = np.sqrt(d2.sum(1)) / np.sqrt((_H - 1.0) ** 2 + (_W - 1.0) ** 2)
        flat = (rowbase + idx[i]).ravel()
        cnt[i] = np.bincount(flat, minlength=_HW * _HW).reshape(_HW, _HW)
    return idx, eucw, cnt


_IDX_NP, _EUCW_NP, _CNT_NP = _build_constants()
_raw = _IDX_NP.reshape(_B * _HW, _NEG).astype(np.int64)
_p64 = (np.arange(_B * _HW, dtype=np.int64) % _CHUNK)[:, None]
_x = _raw & (_HW // 2 - 1)
_packed = ((((_x >> 7) * 64 + _p64) << 7) | (_x & 127)
           | ((_raw < _HW // 2).astype(np.int64) << 31)).astype(np.uint64)
_IDX = np.ascontiguousarray(
    _packed.reshape(-1).astype(np.uint32).view(np.int32))
_EUCW = np.ascontiguousarray((_EUCW_NP * _FACTOR).reshape(_B * _NPB, 1, _PB))
_CNT = _CNT_NP


def _prep_body(z1_ref, z2_ref, imgf_ref, imgp_ref, cnt_ref, euc_ref,
               v_ref, p0_ref):
    z1 = z1_ref[0]
    z2 = z2_ref[0]
    imgf = imgf_ref[...]
    imgp = imgp_ref[...]
    cntb = cnt_ref[0]
    euc = euc_ref[0]

    n1sq = jnp.sum(z1 * z1, axis=0, keepdims=True)
    n2sq = jnp.sum(z2 * z2, axis=0, keepdims=True)

    g = lax.dot_general(z1, z2, (((0,), (0,)), ((), ())),
                        preferred_element_type=jnp.float32)

    s_row = jnp.sum(imgf * imgf, axis=0, keepdims=True)
    m4 = jnp.concatenate([s_row, imgf], axis=0)
    t4 = lax.dot_general(m4, cntb, (((1,), (1,)), ((), ())),
                         preferred_element_type=jnp.float32)
    s_p = jnp.sum(imgp * imgp, axis=0, keepdims=True)
    rgbsq = _NEG * s_p + t4[0:1, :] - 2.0 * jnp.sum(
        imgp * t4[1:4, :], axis=0, keepdims=True)
    rgb = jnp.sqrt(jnp.maximum(rgbsq, 0.0))
    w_row = euc + rgb * ((1.0 - _FACTOR) / np.sqrt(3.0))

    n1col = jnp.transpose(jnp.sqrt(n1sq))
    wcol = jnp.transpose(w_row)
    den = jnp.maximum(n1col * jnp.sqrt(n2sq), _EPS)
    v = jnp.minimum(jnp.abs(g) * wcol / den, 1.0)

    def _bf16_bits(x):
        b = lax.bitcast_convert_type(x, jnp.int32)
        return lax.shift_right_logical(
            b + 0x7FFF + (lax.shift_right_logical(b, 16) & 1), 16)

    for t in range(_HW // 2 // 128):
        lo = _bf16_bits(v[:, t * 128:(t + 1) * 128])
        hi = _bf16_bits(v[:, _HW // 2 + t * 128:_HW // 2 + (t + 1) * 128])
        word = lo | lax.shift_left(hi, 16)
        for p64b in range(_PB // _CHUNK):
            v_ref[0, pl.ds(p64b * 256 + t * _CHUNK, _CHUNK), :] = (
                word[p64b * _CHUNK:(p64b + 1) * _CHUNK, :])

    ratio = jnp.minimum(n1sq / jnp.maximum(n1sq, _EPS), 1.0)
    p0_ref[...] = jnp.full((1, 1, 128), jnp.sum(ratio), jnp.float32)


_prep = pl.pallas_call(
    _prep_body,
    grid=(_B, _NPB),
    in_specs=[
        pl.BlockSpec((1, _C, _PB), lambda i, pb: (i, 0, pb)),
        pl.BlockSpec((1, _C, _HW), lambda i, pb: (i, 0, 0)),
        pl.BlockSpec((3, _HW), lambda i, pb: (0, 0)),
        pl.BlockSpec((3, _PB), lambda i, pb: (0, pb)),
        pl.BlockSpec((1, _PB, _HW), lambda i, pb: (i, pb, 0)),
        pl.BlockSpec((1, 1, _PB), lambda i, pb: (i * _NPB + pb, 0, 0)),
    ],
    out_specs=[
        pl.BlockSpec((1, _HW, 128), lambda i, pb: (i * _NPB + pb, 0, 0)),
        pl.BlockSpec((1, 1, 128), lambda i, pb: (i * _NPB + pb, 0, 0)),
    ],
    out_shape=[
        jax.ShapeDtypeStruct((_B * _NPB, _HW, 128), jnp.int32),
        jax.ShapeDtypeStruct((_B * _NPB, 1, 128), jnp.float32),
    ],
)


_mesh = plsc.VectorSubcoreMesh(core_axis_name="c", subcore_axis_name="s")


_VROWS = _CHUNK * 4
_IB = _CHUNK * _NEG


@functools.partial(
    pl.kernel,
    mesh=_mesh,
    out_type=jax.ShapeDtypeStruct((_NTILES * _NEG,), jnp.float32),
    scratch_types=[
        pltpu.VMEM((2, _VROWS, 128), jnp.int32),
        pltpu.VMEM((2 * _IB,), jnp.int32),
        pltpu.VMEM((_NEG,), jnp.float32),
        pltpu.SemaphoreType.DMA,
        pltpu.SemaphoreType.DMA,
        pltpu.SemaphoreType.DMA,
        pltpu.SemaphoreType.DMA,
    ],
    compiler_params=pltpu.CompilerParams(needs_layout_passes=False),
)
def _sc_gather(v_hbm, idx_hbm, out_hbm, vbuf, ibuf, accbuf,
               sv0, si0, sv1, si1):
    wid = lax.axis_index("s") * 2 + lax.axis_index("c")
    sems = ((sv0, si0), (sv1, si1))

    def copies(ch, slot):
        sv, si = sems[slot]
        return (
            pltpu.make_async_copy(
                v_hbm.at[pl.ds(wid * _HW + ch * _VROWS, _VROWS)],
                vbuf.at[slot], sv),
            pltpu.make_async_copy(
                idx_hbm.at[pl.ds((wid * _RPT + ch * _CHUNK) * _NEG, _IB)],
                ibuf.at[pl.ds(slot * _IB, _IB)], si),
        )

    def issue(ch, slot):
        for c in copies(ch, slot):
            c.start()

    def wait(ch, slot):
        for c in copies(ch, slot):
            c.wait()

    def compute(slot, accs):
        def row_body(r, accs_t):
            accs_l = list(accs_t)
            for k in range(_NVEC):
                pw = ibuf[pl.ds(slot * _IB + r * _NEG + k * 16, 16)]
                qv = lax.shift_right_logical(pw, jnp.int32(7)) & jnp.int32(255)
                cv = pw & jnp.int32(127)
                sh = lax.shift_right_logical(pw, jnp.int32(27))
                w = plsc.load_gather(vbuf.at[slot], [qv, cv])
                fb = lax.shift_left(w, sh) & jnp.int32(-65536)
                accs_l[k] = accs_l[k] + plsc.bitcast(fb, jnp.float32)
            return tuple(accs_l)

        return lax.fori_loop(0, _CHUNK, row_body, accs)

    issue(0, 0)

    def pair_body(c2, accs):
        c0 = 2 * c2
        issue(c0 + 1, 1)
        wait(c0, 0)
        accs = compute(0, accs)

        @pl.when(c2 < _NCHUNKS // 2 - 1)
        def _():
            issue(c0 + 2, 0)

        wait(c0 + 1, 1)
        return compute(1, accs)

    accs = lax.fori_loop(
        0, _NCHUNKS // 2, pair_body,
        tuple(jnp.zeros((16,), jnp.float32) for _ in range(_NVEC)))
    for k in range(_NVEC):
        accbuf[pl.ds(k * 16, 16)] = accs[k]
    pltpu.sync_copy(accbuf, out_hbm.at[pl.ds(wid * _NEG, _NEG)])


def _final_body(sp_ref, p0_ref, out_ref):
    sp = sp_ref[...]
    p0p = p0_ref[...]
    s = jnp.sum(sp, axis=1)
    p0 = jnp.sum(p0p, axis=1)[:, 0:1] / _HW
    pred = s / _HW / _TEMP
    negterm = jnp.maximum(jnp.log(1.0 - pred), -100.0)
    safe = jnp.where(p0 > 0.0, p0, 0.5)
    posterm = jnp.where(p0 > 0.0, jnp.maximum(jnp.log(safe), -100.0), -100.0)
    lossi = -(posterm + jnp.sum(negterm, axis=1, keepdims=True)) / (_NEG + 1.0)
    loss = jnp.sum(lossi) / _B
    o2 = jnp.sum(p0) / _B
    o3 = jnp.sum(pred) / _NEG * _TEMP / _B
    lane = lax.broadcasted_iota(jnp.int32, (1, 128), 1)
    out_ref[...] = jnp.where(
        lane == 0, loss, jnp.where(lane == 1, o2, jnp.where(lane == 2, o3, 0.0)))


_final = pl.pallas_call(
    _final_body,
    out_shape=jax.ShapeDtypeStruct((1, 128), jnp.float32),
)


def kernel(views_1, views_2, img):
    z1 = views_1.reshape(_B, _C, _HW)
    z2 = views_2.reshape(_B, _C, _HW)
    imgf = img.reshape(3, _HW)
    v, p0 = _prep(z1, z2, imgf, imgf, _CNT, _EUCW)
    sp = _sc_gather(v.reshape(_B * _NPB * _HW, 128), _IDX)
    out = _final(sp.reshape(_B, _NTILES // _B, _NEG),
                 p0.reshape(_B, _NPB, 128))
    return (out[0, 0], out[0, 1], out[0, 2])
